# Initial kernel scaffold; baseline (speedup 1.0000x reference)
import jax
import jax.numpy as jnp
from jax.experimental import pallas as pl


def kernel(x, edge_index, edge_attr, W_node1, b_node1, W_edge1, b_edge1, W_msg1, b_msg1, W_node2, b_node2, W_edge2, b_edge2, W_msg2, b_msg2, g1, beta1, g2, beta2, g3, beta3, W_gat, att_src, att_dst, b_gat, W_m1, b_m1, W_m2, b_m2, W_m3, b_m3):
    raise NotImplementedError("write your pallas kernel here")


# speedup vs baseline: 21.0905x; 21.0905x over previous
import functools

import jax
import jax.numpy as jnp
from jax import lax
from jax.experimental import pallas as pl
from jax.experimental.pallas import tpu as pltpu
from jax.experimental.pallas import tpu_sc as plsc

N = 10000
E = 320000
F_IN = 128
H = 64
HEADS = 4

NC = 2
NS = 16
NW = NC * NS
EPW = E // NW
CH = 80
NCHUNK = EPW // CH
RC = 200
NOUT = N // RC
KMAX = (NOUT + NS - 1) // NS

f32 = jnp.float32
i32 = jnp.int32

_MESH = plsc.VectorSubcoreMesh(
    core_axis_name="c", subcore_axis_name="s", num_cores=NC, num_subcores=NS)


def _zero_rows(buf, width):
    zv = jnp.zeros((16,), f32)

    def body(r, _):
        for j in range(width // 16):
            buf[r, pl.ds(j * 16, 16)] = zv
        return 0

    lax.fori_loop(0, buf.shape[0], body, 0)


def _zero_acc(acc, zbuf, s):
    for k in range(KMAX):
        cid = s + NS * k

        @pl.when(cid < NOUT)
        def _():
            r0 = pl.multiple_of(cid * RC, 8)
            pltpu.sync_copy(zbuf, acc.at[pl.ds(r0, RC), :])


def _copy_out(acc, bounce, out, c, s):
    for k in range(KMAX):
        cid = s + NS * k

        @pl.when(cid < NOUT)
        def _():
            r0 = pl.multiple_of(cid * RC, 8)
            pltpu.sync_copy(acc.at[pl.ds(r0, RC), :], bounce)
            pltpu.sync_copy(bounce, out.at[c, pl.ds(r0, RC), :])


@functools.partial(
    pl.kernel,
    out_type=(jax.ShapeDtypeStruct((NC, N, H), f32),
              jax.ShapeDtypeStruct((NC, N, 16), f32)),
    mesh=_MESH,
    scratch_types=[
        pltpu.VMEM((CH,), i32),
        pltpu.VMEM((CH,), i32),
        pltpu.VMEM((CH,), f32),
        pltpu.VMEM((CH, H), f32),
        pltpu.VMEM((CH, 16), f32),
        pltpu.VMEM((RC, H), f32),
        pltpu.VMEM((RC, 16), f32),
        pltpu.VMEM_SHARED((N, H), f32),
        pltpu.VMEM_SHARED((N, 16), f32),
        pltpu.SemaphoreType.DMA,
    ],
    compiler_params=pltpu.CompilerParams(use_tc_tiling_on_sc=False),
)
def _sweep_a(src_h, dst_h, ea_h, tbl_h, out_a, out_s,
             sidx, didx, eav, rows, srows, bw, bn_, acc_a, acc_s, sem):
    c = lax.axis_index("c")
    s = lax.axis_index("s")
    w = c * NS + s
    _zero_rows(bw, H)
    _zero_rows(bn_, 16)
    iota16 = lax.iota(i32, 16)
    m0 = iota16 == 0
    l1v = jnp.where(iota16 == 1, 1.0, 0.0).astype(f32)
    _zero_acc(acc_a, bw, s)
    _zero_acc(acc_s, bn_, s)
    plsc.subcore_barrier()

    base = w * EPW

    def chunk(i, _):
        off = base + i * CH
        pltpu.sync_copy(src_h.at[pl.ds(off, CH)], sidx)
        pltpu.sync_copy(dst_h.at[pl.ds(off, CH)], didx)
        pltpu.sync_copy(ea_h.at[pl.ds(off, CH)], eav)
        pltpu.async_copy(tbl_h.at[sidx], rows, sem).wait()
        for g in range(CH // 16):
            ev = eav[pl.ds(g * 16, 16)]
            for l in range(16):
                srows[g * 16 + l, :] = jnp.where(m0, ev[l], l1v)
        pltpu.sync_copy(rows, acc_a.at[didx], add=True)
        pltpu.sync_copy(srows, acc_s.at[didx], add=True)
        return 0

    lax.fori_loop(0, NCHUNK, chunk, 0)
    plsc.subcore_barrier()
    _copy_out(acc_a, bw, out_a, c, s)
    _copy_out(acc_s, bn_, out_s, c, s)


@functools.partial(
    pl.kernel,
    out_type=jax.ShapeDtypeStruct((NC, N, H), f32),
    mesh=_MESH,
    scratch_types=[
        pltpu.VMEM((CH,), i32),
        pltpu.VMEM((CH,), i32),
        pltpu.VMEM((CH, H), f32),
        pltpu.VMEM((RC, H), f32),
        pltpu.VMEM_SHARED((N, H), f32),
        pltpu.SemaphoreType.DMA,
    ],
    compiler_params=pltpu.CompilerParams(use_tc_tiling_on_sc=False),
)
def _sweep_b(src_h, dst_h, tbl_h, out_h, sidx, didx, rows, bw, acc, sem):
    c = lax.axis_index("c")
    s = lax.axis_index("s")
    w = c * NS + s
    _zero_rows(bw, H)
    _zero_acc(acc, bw, s)
    plsc.subcore_barrier()

    base = w * EPW

    def chunk(i, _):
        off = base + i * CH
        pltpu.sync_copy(src_h.at[pl.ds(off, CH)], sidx)
        pltpu.sync_copy(dst_h.at[pl.ds(off, CH)], didx)
        pltpu.async_copy(tbl_h.at[sidx], rows, sem).wait()
        pltpu.sync_copy(rows, acc.at[didx], add=True)
        return 0

    lax.fori_loop(0, NCHUNK, chunk, 0)
    plsc.subcore_barrier()
    _copy_out(acc, bw, out_h, c, s)


@functools.partial(
    pl.kernel,
    out_type=jax.ShapeDtypeStruct((NC, N, 16), f32),
    mesh=_MESH,
    scratch_types=[
        pltpu.VMEM((CH,), i32),
        pltpu.VMEM((CH,), i32),
        pltpu.VMEM((CH, 16), f32),
        pltpu.VMEM((CH, 32), f32),
        pltpu.VMEM((CH, 16), f32),
        pltpu.VMEM((RC, 16), f32),
        pltpu.VMEM_SHARED((N, 16), f32),
        pltpu.SemaphoreType.DMA,
    ],
    compiler_params=pltpu.CompilerParams(use_tc_tiling_on_sc=False),
)
def _sweep_c(src_h, dst_h, gs_h, gd_h, out_h,
             sidx, didx, gsr, gdr, exr, bn_, acc, sem):
    c = lax.axis_index("c")
    s = lax.axis_index("s")
    w = c * NS + s
    _zero_rows(bn_, 16)
    _zero_acc(acc, bn_, s)
    plsc.subcore_barrier()

    iota16 = lax.iota(i32, 16)
    lane_m = iota16 < 4
    base = w * EPW

    def chunk(i, _):
        off = base + i * CH
        pltpu.sync_copy(src_h.at[pl.ds(off, CH)], sidx)
        pltpu.sync_copy(dst_h.at[pl.ds(off, CH)], didx)
        pltpu.async_copy(gs_h.at[sidx], gsr, sem).wait()
        pltpu.async_copy(gd_h.at[didx], gdr, sem).wait()

        def row(r, _2):
            vs = gsr[r, :]
            va = gdr[r, pl.ds(0, 16)]
            ve = gdr[r, pl.ds(4, 16)]
            t = vs + va
            lk = jnp.where(t >= 0.0, t, 0.2 * t)
            ex = jnp.exp(lk - ve)
            exr[r, :] = jnp.where(lane_m, ex, 0.0)
            return 0

        lax.fori_loop(0, CH, row, 0)
        pltpu.sync_copy(exr, acc.at[didx], add=True)
        return 0

    lax.fori_loop(0, NCHUNK, chunk, 0)
    plsc.subcore_barrier()
    _copy_out(acc, bn_, out_h, c, s)


@functools.partial(
    pl.kernel,
    out_type=jax.ShapeDtypeStruct((NC, N, H), f32),
    mesh=_MESH,
    scratch_types=[
        pltpu.VMEM((CH,), i32),
        pltpu.VMEM((CH,), i32),
        pltpu.VMEM((CH, 16), f32),
        pltpu.VMEM((CH, 32), f32),
        pltpu.VMEM((CH, HEADS * H), f32),
        pltpu.VMEM((CH, H), f32),
        pltpu.VMEM((RC, H), f32),
        pltpu.VMEM_SHARED((N, H), f32),
        pltpu.SemaphoreType.DMA,
    ],
    compiler_params=pltpu.CompilerParams(use_tc_tiling_on_sc=False),
)
def _sweep_d(src_h, dst_h, g_h, gs_h, gd2_h, out_h,
             sidx, didx, gsr, g2r, grows, msg, bw, acc, sem):
    c = lax.axis_index("c")
    s = lax.axis_index("s")
    w = c * NS + s
    _zero_rows(bw, H)
    _zero_acc(acc, bw, s)
    plsc.subcore_barrier()

    base = w * EPW

    def chunk(i, _):
        off = base + i * CH
        pltpu.sync_copy(src_h.at[pl.ds(off, CH)], sidx)
        pltpu.sync_copy(dst_h.at[pl.ds(off, CH)], didx)
        pltpu.async_copy(g_h.at[sidx], grows, sem).wait()
        pltpu.async_copy(gs_h.at[sidx], gsr, sem).wait()
        pltpu.async_copy(gd2_h.at[didx], g2r, sem).wait()

        def row(r, _2):
            vs = gsr[r, :]
            va = g2r[r, pl.ds(0, 16)]
            ve = g2r[r, pl.ds(4, 16)]
            vq = g2r[r, pl.ds(8, 16)]
            t = vs + va
            lk = jnp.where(t >= 0.0, t, 0.2 * t)
            wv = jnp.exp(lk - ve) * vq
            for j in range(H // 16):
                v = wv[0] * grows[r, pl.ds(j * 16, 16)]
                for hd in range(1, HEADS):
                    v = v + wv[hd] * grows[r, pl.ds(hd * H + j * 16, 16)]
                msg[r, pl.ds(j * 16, 16)] = v
            return 0

        lax.fori_loop(0, CH, row, 0)
        pltpu.sync_copy(msg, acc.at[didx], add=True)
        return 0

    lax.fori_loop(0, NCHUNK, chunk, 0)
    plsc.subcore_barrier()
    _copy_out(acc, bw, out_h, c, s)


def _t1_body(x_ref, a1_ref, wn_ref, bn1_ref, y_ref, nf_ref):
    x = x_ref[...]
    y_ref[...] = jnp.dot(x, a1_ref[...], preferred_element_type=f32)
    nf_ref[...] = jnp.dot(x, wn_ref[...], preferred_element_type=f32) + bn1_ref[...]


def _mp_tail(acc_ref, accs_ref, y_ref, nf_ref, we_ref, be_ref, bb_ref, bm_ref,
             g_ref, beta_ref):
    a = acc_ref[...]
    sv = a[0] + a[1]
    ssc = accs_ref[...]
    sums = ssc[0] + ssc[1]
    sum_ea = sums[:, 0:1]
    deg = sums[:, 1:2]
    bb = bb_ref[...]
    u = jnp.dot(we_ref[...], bb, preferred_element_type=f32)
    cc = jnp.dot(be_ref[...], bb, preferred_element_type=f32) + bm_ref[...]
    y = y_ref[...]
    aggr = sv + sum_ea * u + deg * cc + y + cc + nf_ref[...]
    h = jnp.maximum(aggr, 0.0)
    mu = jnp.mean(h, 0, keepdims=True)
    var = jnp.mean((h - mu) ** 2, 0, keepdims=True)
    hn = g_ref[...] * (h - mu) * lax.rsqrt(var + 1e-5) + beta_ref[...]
    return jnp.maximum(hn, 0.0)


def _t2_body(acc_ref, accs_ref, y1_ref, nf1_ref, we1_ref, be1_ref, b1_ref,
             bm1_ref, g1_ref, beta1_ref, wn2_ref, bn2_ref, a2_ref,
             y2_ref, nf2_ref):
    h = _mp_tail(acc_ref, accs_ref, y1_ref, nf1_ref, we1_ref, be1_ref, b1_ref,
                 bm1_ref, g1_ref, beta1_ref)
    y2_ref[...] = jnp.dot(h, a2_ref[...], preferred_element_type=f32)
    nf2_ref[...] = jnp.dot(h, wn2_ref[...], preferred_element_type=f32) + bn2_ref[...]


def _t3_body(acc_ref, accs_ref, y2_ref, nf2_ref, we2_ref, be2_ref, b2_ref,
             bm2_ref, g2_ref, beta2_ref, wg_ref, as_ref, ad_ref,
             g_out_ref, gs_ref, gd_ref):
    h = _mp_tail(acc_ref, accs_ref, y2_ref, nf2_ref, we2_ref, be2_ref, b2_ref,
                 bm2_ref, g2_ref, beta2_ref)
    hg = jnp.dot(h, wg_ref[...], preferred_element_type=f32)
    g_out_ref[...] = hg
    a_s = jnp.dot(hg, as_ref[...], preferred_element_type=f32)
    a_d = jnp.dot(hg, ad_ref[...], preferred_element_type=f32)
    t = a_s + a_d
    el = jnp.where(t >= 0.0, t, 0.2 * t)
    gs_ref[...] = jnp.concatenate([a_s, jnp.zeros((N, 12), f32)], axis=1)
    gd_ref[...] = jnp.concatenate([a_d, el, jnp.zeros((N, 24), f32)], axis=1)


def _t4_body(acce_ref, gd_ref, gd2_ref):
    a = acce_ref[...]
    ssum = a[0, :, 0:4] + a[1, :, 0:4]
    q = 1.0 / (4.0 * (ssum + 1.0 + 1e-16))
    gd2_ref[...] = jnp.concatenate(
        [gd_ref[...][:, 0:8], q, jnp.zeros((N, 20), f32)], axis=1)


def _t5_body(accg_ref, g_ref, gd2_ref, exp_ref, bg_ref, g3_ref, beta3_ref,
             w1_ref, b1_ref, w2_ref, b2_ref, w3_ref, b3_ref, out_ref):
    gv = g_ref[...]
    q = gd2_ref[...][:, 8:12]
    wsum = gv * jnp.dot(q, exp_ref[...], preferred_element_type=f32)
    self_t = (wsum[:, 0:64] + wsum[:, 64:128]
              + wsum[:, 128:192] + wsum[:, 192:256])
    a = accg_ref[...]
    gat = a[0] + a[1] + self_t + bg_ref[...]
    mu = jnp.mean(gat, 0, keepdims=True)
    var = jnp.mean((gat - mu) ** 2, 0, keepdims=True)
    h = jnp.maximum(
        g3_ref[...] * (gat - mu) * lax.rsqrt(var + 1e-5) + beta3_ref[...], 0.0)
    h = jnp.maximum(jnp.dot(h, w1_ref[...], preferred_element_type=f32)
                    + b1_ref[...], 0.0)
    h = jnp.maximum(jnp.dot(h, w2_ref[...], preferred_element_type=f32)
                    + b2_ref[...], 0.0)
    out_ref[...] = jnp.dot(h, w3_ref[...], preferred_element_type=f32) + b3_ref[...]


def _sds(*shape):
    return jax.ShapeDtypeStruct(shape, f32)


def kernel(x, edge_index, edge_attr, W_node1, b_node1, W_edge1, b_edge1,
           W_msg1, b_msg1, W_node2, b_node2, W_edge2, b_edge2, W_msg2, b_msg2,
           g1, beta1, g2, beta2, g3, beta3, W_gat, att_src, att_dst, b_gat,
           W_m1, b_m1, W_m2, b_m2, W_m3, b_m3):
    src = edge_index[0]
    dst = edge_index[1]
    ea = edge_attr[:, 0]
    row = lambda v: v.reshape(1, -1)

    a1 = W_msg1[:F_IN]
    b1w = W_msg1[F_IN:]
    a2 = W_msg2[:H]
    b2w = W_msg2[H:]
    eye = jnp.eye(HEADS, dtype=f32)
    as_m = (att_src[:, :, None] * eye[:, None, :]).reshape(HEADS * H, HEADS)
    ad_m = (att_dst[:, :, None] * eye[:, None, :]).reshape(HEADS * H, HEADS)
    expand = jnp.repeat(eye, H, axis=1)

    y1, nf1 = pl.pallas_call(
        _t1_body, out_shape=(_sds(N, H), _sds(N, H)))(
            x, a1, W_node1, row(b_node1))

    acc_a, acc_s = _sweep_a(src, dst, ea, y1)

    y2, nf2 = pl.pallas_call(
        _t2_body, out_shape=(_sds(N, H), _sds(N, H)))(
            acc_a, acc_s, y1, nf1, W_edge1, row(b_edge1), b1w, row(b_msg1),
            row(g1), row(beta1), W_node2, row(b_node2), a2)

    acc_b = _sweep_b(src, dst, y2)

    g_tbl, gs_tbl, gd_tbl = pl.pallas_call(
        _t3_body, out_shape=(_sds(N, HEADS * H), _sds(N, 16), _sds(N, 32)))(
            acc_b, acc_s, y2, nf2, W_edge2, row(b_edge2), b2w, row(b_msg2),
            row(g2), row(beta2), W_gat, as_m, ad_m)

    acc_e = _sweep_c(src, dst, gs_tbl, gd_tbl)

    gd2_tbl = pl.pallas_call(_t4_body, out_shape=_sds(N, 32))(acc_e, gd_tbl)

    acc_g = _sweep_d(src, dst, g_tbl, gs_tbl, gd2_tbl)

    wear = pl.pallas_call(
        _t5_body, out_shape=_sds(N, 1))(
            acc_g, g_tbl, gd2_tbl, expand, row(b_gat), row(g3), row(beta3),
            W_m1, row(b_m1), W_m2, row(b_m2), W_m3, row(b_m3))

    return wear.reshape(-1)
